# Initial kernel scaffold; baseline (speedup 1.0000x reference)
import jax
import jax.numpy as jnp
from jax.experimental import pallas as pl


def kernel(x, conv_b, conv_w, bn_gamma, bn_beta):
    raise NotImplementedError("write your pallas kernel here")


# speedup vs baseline: 5.4857x; 5.4857x over previous
import jax
import jax.numpy as jnp
from jax import lax
from jax.experimental import pallas as pl
from jax.experimental.pallas import tpu as pltpu

_BN_EPS = 1e-5
_VMEM_LIMIT = 32 * 1024 * 1024


def _patches3(x):
    xb = x.astype(jnp.bfloat16)
    z = jnp.zeros((xb.shape[0], 1), jnp.bfloat16)
    xm = jnp.concatenate([z, xb[:, :-1]], axis=1)
    xp = jnp.concatenate([xb[:, 1:], z], axis=1)
    return jnp.concatenate([xm, xb, xp], axis=0)


def _stats_kernel(x_ref, w_ref, stats_ref):
    p = _patches3(x_ref[0])
    y = jnp.dot(w_ref[...], p, preferred_element_type=jnp.float32)
    s = jnp.sum(y, axis=1, keepdims=True)
    ss = jnp.sum(y * y, axis=1, keepdims=True)
    stats_ref[0] = jnp.concatenate([s, ss], axis=1)


def _apply_kernel(x_ref, w_ref, shift_ref, o_ref):
    p = _patches3(x_ref[0])
    y = jnp.dot(w_ref[...], p, preferred_element_type=jnp.float32)
    o_ref[0] = jnp.maximum(y + shift_ref[...], 0.0)


def kernel(x, conv_b, conv_w, bn_gamma, bn_beta):
    del conv_b
    n, c_in, l = x.shape
    c_out = conv_w.shape[0]
    m = n * l
    ck = 3 * c_in
    w_flat = conv_w.transpose(0, 2, 1).reshape(c_out, ck)

    params = pltpu.CompilerParams(
        dimension_semantics=("parallel",), vmem_limit_bytes=_VMEM_LIMIT)

    stats = pl.pallas_call(
        _stats_kernel,
        out_shape=jax.ShapeDtypeStruct((n, c_out, 2), jnp.float32),
        grid=(n,),
        in_specs=[
            pl.BlockSpec((1, c_in, l), lambda i: (i, 0, 0)),
            pl.BlockSpec((c_out, ck), lambda i: (0, 0)),
        ],
        out_specs=pl.BlockSpec((1, c_out, 2), lambda i: (i, 0, 0)),
        compiler_params=params,
    )(x, w_flat.astype(jnp.bfloat16))

    tot = jnp.sum(stats, axis=0)
    mean = tot[:, 0] / m
    var = jnp.maximum(tot[:, 1] / m - mean * mean, 0.0)
    scale = bn_gamma * lax.rsqrt(var + _BN_EPS)
    shift = (bn_beta - mean * scale).reshape(c_out, 1)
    w_scaled = (w_flat * scale[:, None]).astype(jnp.bfloat16)

    return pl.pallas_call(
        _apply_kernel,
        out_shape=jax.ShapeDtypeStruct((n, c_out, l), jnp.float32),
        grid=(n,),
        in_specs=[
            pl.BlockSpec((1, c_in, l), lambda i: (i, 0, 0)),
            pl.BlockSpec((c_out, ck), lambda i: (0, 0)),
            pl.BlockSpec((c_out, 1), lambda i: (0, 0)),
        ],
        out_specs=pl.BlockSpec((1, c_out, l), lambda i: (i, 0, 0)),
        compiler_params=params,
    )(x, w_scaled, shift)


# speedup vs baseline: 6.0291x; 1.0991x over previous
import jax
import jax.numpy as jnp
from jax import lax
from jax.experimental import pallas as pl
from jax.experimental.pallas import tpu as pltpu

_BN_EPS = 1e-5
_VMEM_LIMIT = 32 * 1024 * 1024


def _patches3(xb):
    z = jnp.zeros((xb.shape[0], 1), jnp.bfloat16)
    xm = jnp.concatenate([z, xb[:, :-1]], axis=1)
    xp = jnp.concatenate([xb[:, 1:], z], axis=1)
    return jnp.concatenate([xm, xb, xp], axis=0)


def _stats_kernel(x_ref, w_ref, xb_ref, stats_ref):
    i = pl.program_id(0)

    @pl.when(i == 0)
    def _():
        stats_ref[...] = jnp.zeros_like(stats_ref)

    xb = x_ref[0].astype(jnp.bfloat16)
    xb_ref[0] = xb
    p = _patches3(xb)
    y = jnp.dot(w_ref[...], p, preferred_element_type=jnp.float32)
    s = jnp.sum(y, axis=1, keepdims=True)
    ss = jnp.sum(y * y, axis=1, keepdims=True)
    stats_ref[...] += jnp.concatenate([s, ss], axis=1)


def _apply_kernel(xb_ref, w_ref, shift_ref, o_ref):
    p = _patches3(xb_ref[0])
    y = jnp.dot(w_ref[...], p, preferred_element_type=jnp.float32)
    o_ref[0] = jnp.maximum(y + shift_ref[...], 0.0)


def kernel(x, conv_b, conv_w, bn_gamma, bn_beta):
    del conv_b
    n, c_in, l = x.shape
    c_out = conv_w.shape[0]
    m = n * l
    ck = 3 * c_in
    w_flat = conv_w.transpose(0, 2, 1).reshape(c_out, ck)

    params_seq = pltpu.CompilerParams(
        dimension_semantics=("arbitrary",), vmem_limit_bytes=_VMEM_LIMIT)
    params = pltpu.CompilerParams(
        dimension_semantics=("parallel",), vmem_limit_bytes=_VMEM_LIMIT)

    xb, tot = pl.pallas_call(
        _stats_kernel,
        out_shape=(
            jax.ShapeDtypeStruct((n, c_in, l), jnp.bfloat16),
            jax.ShapeDtypeStruct((c_out, 2), jnp.float32),
        ),
        grid=(n,),
        in_specs=[
            pl.BlockSpec((1, c_in, l), lambda i: (i, 0, 0)),
            pl.BlockSpec((c_out, ck), lambda i: (0, 0)),
        ],
        out_specs=(
            pl.BlockSpec((1, c_in, l), lambda i: (i, 0, 0)),
            pl.BlockSpec((c_out, 2), lambda i: (0, 0)),
        ),
        compiler_params=params_seq,
    )(x, w_flat.astype(jnp.bfloat16))

    mean = tot[:, 0] / m
    var = jnp.maximum(tot[:, 1] / m - mean * mean, 0.0)
    scale = bn_gamma * lax.rsqrt(var + _BN_EPS)
    shift = (bn_beta - mean * scale).reshape(c_out, 1)
    w_scaled = (w_flat * scale[:, None]).astype(jnp.bfloat16)

    return pl.pallas_call(
        _apply_kernel,
        out_shape=jax.ShapeDtypeStruct((n, c_out, l), jnp.float32),
        grid=(n,),
        in_specs=[
            pl.BlockSpec((1, c_in, l), lambda i: (i, 0, 0)),
            pl.BlockSpec((c_out, ck), lambda i: (0, 0)),
            pl.BlockSpec((c_out, 1), lambda i: (0, 0)),
        ],
        out_specs=pl.BlockSpec((1, c_out, l), lambda i: (i, 0, 0)),
        compiler_params=params,
    )(xb, w_scaled, shift)
